# baseline (speedup 1.0000x reference)
import functools

import jax
import jax.numpy as jnp
from jax import lax
from jax.experimental import pallas as pl
from jax.experimental.pallas import tpu as pltpu
from jax.experimental.pallas import tpu_sc as plsc

N_EMBD = 64
VOCAB = 1000000
NW = 32
VB = 128
NPACK = VOCAB // 2
N_FULL = VOCAB // VB
MAIN_FULL = N_FULL // NW
TAIL_V0 = N_FULL * VB
LANES = 16

_mesh = plsc.VectorSubcoreMesh(core_axis_name="c", subcore_axis_name="s")
_params = pltpu.CompilerParams(
    use_tc_tiling_on_sc=True, needs_layout_passes=False)


def _iota16():
    return lax.iota(jnp.int32, 16)


@functools.partial(
    pl.kernel,
    mesh=_mesh,
    compiler_params=_params,
    out_type=jax.ShapeDtypeStruct((NPACK, 128), jnp.float32),
    scratch_types=[
        pltpu.VMEM((2, N_EMBD, VB), jnp.float32),
        pltpu.VMEM((2, N_EMBD, VB), jnp.float32),
        pltpu.VMEM((N_EMBD, N_EMBD), jnp.float32),
        pltpu.VMEM((N_EMBD // 2, VB), jnp.float32),
        pltpu.SemaphoreType.DMA,
        pltpu.SemaphoreType.DMA,
        pltpu.SemaphoreType.DMA,
        pltpu.SemaphoreType.DMA,
    ],
)
def _pack_table(tT_hbm, packed_hbm, tin, tout, tin_x, tout_x,
                gsem0, gsem1, osem0, osem1):
    gsem = [gsem0, gsem1]
    osem = [osem0, osem1]
    wid = lax.axis_index("s") * 2 + lax.axis_index("c")
    blk0 = wid * MAIN_FULL

    def fire_load(j, r):
        pltpu.async_copy(
            tT_hbm.at[:, pl.ds((blk0 + j) * VB, VB)], tin.at[r], gsem[r])

    def transpose_block(src, dst, npairs):
        for p in range(npairs):
            for h in range(2):
                cols = jnp.full((16,), 2 * p + h, jnp.int32)
                for g in range(4):
                    rows = _iota16() + g * 16
                    dst[p, pl.ds(h * 64 + g * 16, 16)] = plsc.load_gather(
                        src, [rows, cols])

    def process(j, r):
        pltpu.make_async_copy(
            tT_hbm.at[:, pl.ds(0, VB)], tin.at[r], gsem[r]).wait()
        transpose_block(tin.at[r], tout.at[r], N_EMBD)
        pltpu.async_copy(
            tout.at[r], packed_hbm.at[pl.ds((blk0 + j) * (VB // 2), VB // 2)],
            osem[r])

    def drain_out(r):
        pltpu.make_async_copy(
            tout.at[r], packed_hbm.at[pl.ds(0, VB // 2)], osem[r]).wait()

    fire_load(0, 0)

    def outer(it, carry):
        j0 = it * 2
        process(j0, 0)

        @pl.when(it > 0)
        def _():
            drain_out(1)

        fire_load(j0 + 1, 1)
        process(j0 + 1, 1)

        @pl.when(it < MAIN_FULL // 2 - 1)
        def _():
            drain_out(0)
            fire_load(j0 + 2, 0)

        return carry

    lax.fori_loop(0, MAIN_FULL // 2, outer, 0)
    drain_out(0)
    drain_out(1)

    @pl.when(wid < 4)
    def _():
        blk = NW * MAIN_FULL + wid
        pltpu.sync_copy(tT_hbm.at[:, pl.ds(blk * VB, VB)], tin.at[0])
        transpose_block(tin.at[0], tout.at[0], N_EMBD)
        pltpu.sync_copy(
            tout.at[0], packed_hbm.at[pl.ds(blk * (VB // 2), VB // 2)])

    @pl.when(wid == 4)
    def _():
        pltpu.sync_copy(tT_hbm.at[:, pl.ds(TAIL_V0, N_EMBD)], tin_x)
        transpose_block(tin_x, tout_x, N_EMBD // 2)
        pltpu.sync_copy(
            tout_x, packed_hbm.at[pl.ds(TAIL_V0 // 2, N_EMBD // 2)])


@functools.partial(
    pl.kernel,
    mesh=_mesh,
    compiler_params=_params,
    out_type=jax.ShapeDtypeStruct((32, 32, N_EMBD, 1024), jnp.float32),
    scratch_types=[
        pltpu.VMEM((32, 1024), jnp.int32),
        pltpu.VMEM((2, 128), jnp.int32),
        pltpu.VMEM((2, 128, 128), jnp.float32),
        pltpu.VMEM((2, N_EMBD, 128), jnp.float32),
        pltpu.VMEM((32, N_EMBD), jnp.float32),
        pltpu.SemaphoreType.DMA,
        pltpu.SemaphoreType.DMA,
        pltpu.SemaphoreType.DMA,
        pltpu.SemaphoreType.DMA,
    ],
)
def _gather(xT_hbm, packed_hbm, pos_hbm, outT_hbm, idx_v, pair_v, rows_v,
            tout, pos_v, gsem0, gsem1, osem0, osem1):
    gsem = [gsem0, gsem1]
    osem = [osem0, osem1]
    wid = lax.axis_index("s") * 2 + lax.axis_index("c")
    pltpu.sync_copy(xT_hbm.at[wid], idx_v)
    pltpu.sync_copy(pos_hbm, pos_v)
    NBLK = 256

    def fire_gather(i, r):
        u = i // 8
        bb = i % 8
        for g in range(8):
            iv = idx_v[u, pl.ds(bb * 128 + g * 16, 16)]
            pair_v[r, pl.ds(g * 16, 16)] = lax.shift_right_logical(iv, 1)
        pltpu.async_copy(
            packed_hbm.at[pair_v.at[r]], rows_v.at[r], gsem[r])

    def process(i, r):
        u = i // 8
        bb = i % 8
        pltpu.make_async_copy(
            packed_hbm.at[pl.ds(0, 128)], rows_v.at[r], gsem[r]).wait()
        pf = []
        for g in range(8):
            iv = idx_v[u, pl.ds(bb * 128 + g * 16, 16)]
            pf.append(lax.shift_left(jnp.bitwise_and(iv, 1), 6))

        def erow(e8, carry):
            for ee in range(8):
                e = e8 * 8 + ee
                ps = plsc.load_gather(
                    pos_v, [jnp.full((16,), u, jnp.int32),
                            jnp.full((16,), e, jnp.int32)])
                for g in range(8):
                    rows = _iota16() + g * 16
                    vals = plsc.load_gather(rows_v.at[r], [rows, pf[g] + e])
                    tout[r, e, pl.ds(g * 16, 16)] = vals + ps
            return carry

        lax.fori_loop(0, 8, erow, 0)
        pltpu.async_copy(
            tout.at[r], outT_hbm.at[wid, u, :, pl.ds(bb * 128, 128)],
            osem[r])

    def drain_out(r):
        pltpu.make_async_copy(
            tout.at[r], outT_hbm.at[0, 0, :, pl.ds(0, 128)], osem[r]).wait()

    fire_gather(0, 0)

    def outer(it, carry):
        i0 = it * 2
        process(i0, 0)

        @pl.when(it > 0)
        def _():
            drain_out(1)

        fire_gather(i0 + 1, 1)
        process(i0 + 1, 1)

        @pl.when(it < NBLK // 2 - 1)
        def _():
            drain_out(0)
            fire_gather(i0 + 2, 0)

        return carry

    lax.fori_loop(0, NBLK // 2, outer, 0)
    drain_out(0)
    drain_out(1)


def kernel(x, token_table, pos_table):
    tT = token_table.T
    xT = jnp.transpose(x, (1, 2, 0))
    packed = _pack_table(tT)
    outT = _gather(xT, packed, pos_table)
    return jnp.transpose(outT, (3, 0, 1, 2))


# speedup vs baseline: 1.5908x; 1.5908x over previous
import functools

import jax
import jax.numpy as jnp
from jax import lax
from jax.experimental import pallas as pl
from jax.experimental.pallas import tpu as pltpu
from jax.experimental.pallas import tpu_sc as plsc

N_EMBD = 64
VOCAB = 1000000
NW = 32
VB = 128
NPACK = VOCAB // 2
N_FULL = VOCAB // VB
MAIN_FULL = N_FULL // NW
TAIL_V0 = N_FULL * VB
LANES = 16

_mesh = plsc.VectorSubcoreMesh(core_axis_name="c", subcore_axis_name="s")
_params = pltpu.CompilerParams(
    use_tc_tiling_on_sc=True, needs_layout_passes=False)


def _iota16():
    return lax.iota(jnp.int32, 16)


@functools.partial(
    pl.kernel,
    mesh=_mesh,
    compiler_params=_params,
    out_type=jax.ShapeDtypeStruct((NPACK, 128), jnp.float32),
    scratch_types=[
        pltpu.VMEM((2, N_EMBD, VB), jnp.float32),
        pltpu.VMEM((2, N_EMBD, VB), jnp.float32),
        pltpu.VMEM((N_EMBD, N_EMBD), jnp.float32),
        pltpu.VMEM((N_EMBD // 2, VB), jnp.float32),
        pltpu.SemaphoreType.DMA,
        pltpu.SemaphoreType.DMA,
        pltpu.SemaphoreType.DMA,
        pltpu.SemaphoreType.DMA,
    ],
)
def _pack_table(tT_hbm, packed_hbm, tin, tout, tin_x, tout_x,
                gsem0, gsem1, osem0, osem1):
    gsem = [gsem0, gsem1]
    osem = [osem0, osem1]
    wid = lax.axis_index("s") * 2 + lax.axis_index("c")
    blk0 = wid * MAIN_FULL

    def fire_load(j, r):
        pltpu.async_copy(
            tT_hbm.at[:, pl.ds((blk0 + j) * VB, VB)], tin.at[r], gsem[r])

    def wait_load(r):
        pltpu.make_async_copy(
            tT_hbm.at[:, pl.ds(0, VB)], tin.at[r], gsem[r]).wait()

    def transpose_pairs(src, dst, p0):
        vals = []
        for p in (p0, p0 + 1):
            for h in range(2):
                cols = jnp.full((16,), 2 * p + h, jnp.int32)
                for g in range(4):
                    vals.append(plsc.load_gather(
                        src, [_iota16() + g * 16, cols]))
        k = 0
        for p in (p0, p0 + 1):
            for h in range(2):
                for g in range(4):
                    dst[p, pl.ds(h * 64 + g * 16, 16)] = vals[k]
                    k += 1

    def transpose_block(src, dst, npairs):
        for p0 in range(0, npairs, 2):
            transpose_pairs(src, dst, p0)

    def transpose_block_dyn(src, dst, npairs):
        def body(p, carry):
            vals = []
            for h in range(2):
                cols = jnp.full((16,), 2 * p + h, jnp.int32)
                for g in range(4):
                    vals.append(plsc.load_gather(
                        src, [_iota16() + g * 16, cols]))
            for k in range(8):
                h, g = divmod(k, 4)
                dst[p, pl.ds(h * 64 + g * 16, 16)] = vals[k]
            return carry

        lax.fori_loop(0, npairs, body, 0)

    def process(j, r):
        wait_load(r)
        transpose_block(tin.at[r], tout.at[r], N_EMBD)
        pltpu.async_copy(
            tout.at[r], packed_hbm.at[pl.ds((blk0 + j) * (VB // 2), VB // 2)],
            osem[r])

    def drain_out(r):
        pltpu.make_async_copy(
            tout.at[r], packed_hbm.at[pl.ds(0, VB // 2)], osem[r]).wait()

    fire_load(0, 0)

    def outer(it, carry):
        j0 = it * 2
        fire_load(j0 + 1, 1)

        @pl.when(it > 0)
        def _():
            drain_out(0)

        process(j0, 0)

        @pl.when(it < MAIN_FULL // 2 - 1)
        def _():
            fire_load(j0 + 2, 0)

        @pl.when(it > 0)
        def _():
            drain_out(1)

        process(j0 + 1, 1)
        return carry

    lax.fori_loop(0, MAIN_FULL // 2, outer, 0)
    drain_out(0)
    drain_out(1)

    @pl.when(wid < 4)
    def _():
        blk = NW * MAIN_FULL + wid
        pltpu.sync_copy(tT_hbm.at[:, pl.ds(blk * VB, VB)], tin.at[0])
        transpose_block_dyn(tin.at[0], tout.at[0], N_EMBD)
        pltpu.sync_copy(
            tout.at[0], packed_hbm.at[pl.ds(blk * (VB // 2), VB // 2)])

    @pl.when(wid == 4)
    def _():
        pltpu.sync_copy(tT_hbm.at[:, pl.ds(TAIL_V0, N_EMBD)], tin_x)
        transpose_block_dyn(tin_x, tout_x, N_EMBD // 2)
        pltpu.sync_copy(
            tout_x, packed_hbm.at[pl.ds(TAIL_V0 // 2, N_EMBD // 2)])


@functools.partial(
    pl.kernel,
    mesh=_mesh,
    compiler_params=_params,
    out_type=jax.ShapeDtypeStruct((32, 32, N_EMBD, 1024), jnp.float32),
    scratch_types=[
        pltpu.VMEM((32, 1024), jnp.int32),
        pltpu.VMEM((2, 128), jnp.int32),
        pltpu.VMEM((2, 128, 128), jnp.float32),
        pltpu.VMEM((2, N_EMBD, 128), jnp.float32),
        pltpu.VMEM((32, N_EMBD), jnp.float32),
        pltpu.SemaphoreType.DMA,
        pltpu.SemaphoreType.DMA,
        pltpu.SemaphoreType.DMA,
        pltpu.SemaphoreType.DMA,
    ],
)
def _gather(xT_hbm, packed_hbm, pos_hbm, outT_hbm, idx_v, pair_v, rows_v,
            tout, pos_v, gsem0, gsem1, osem0, osem1):
    gsem = [gsem0, gsem1]
    osem = [osem0, osem1]
    wid = lax.axis_index("s") * 2 + lax.axis_index("c")
    pltpu.sync_copy(xT_hbm.at[wid], idx_v)
    pltpu.sync_copy(pos_hbm, pos_v)
    NBLK = 256

    def fire_gather(i, r):
        u = i // 8
        bb = i % 8
        ivs = [idx_v[u, pl.ds(bb * 128 + g * 16, 16)] for g in range(8)]
        for g in range(8):
            pair_v[r, pl.ds(g * 16, 16)] = lax.shift_right_logical(ivs[g], 1)
        pltpu.async_copy(
            packed_hbm.at[pair_v.at[r]], rows_v.at[r], gsem[r])

    def process(i, r):
        u = i // 8
        bb = i % 8
        pltpu.make_async_copy(
            packed_hbm.at[pl.ds(0, 128)], rows_v.at[r], gsem[r]).wait()
        ivs = [idx_v[u, pl.ds(bb * 128 + g * 16, 16)] for g in range(8)]
        pf = [lax.shift_left(jnp.bitwise_and(iv, 1), 6) for iv in ivs]
        us = jnp.full((16,), u, jnp.int32)
        for e in range(N_EMBD):
            ps = plsc.load_gather(
                pos_v, [us, jnp.full((16,), e, jnp.int32)])
            vs = [plsc.load_gather(rows_v.at[r],
                                   [_iota16() + g * 16, pf[g] + e])
                  for g in range(8)]
            for g in range(8):
                tout[r, e, pl.ds(g * 16, 16)] = vs[g] + ps
        pltpu.async_copy(
            tout.at[r], outT_hbm.at[wid, u, :, pl.ds(bb * 128, 128)],
            osem[r])

    def drain_out(r):
        pltpu.make_async_copy(
            tout.at[r], outT_hbm.at[0, 0, :, pl.ds(0, 128)], osem[r]).wait()

    fire_gather(0, 0)

    def outer(it, carry):
        i0 = it * 2
        fire_gather(i0 + 1, 1)

        @pl.when(it > 0)
        def _():
            drain_out(0)

        process(i0, 0)

        @pl.when(it < NBLK // 2 - 1)
        def _():
            fire_gather(i0 + 2, 0)

        @pl.when(it > 0)
        def _():
            drain_out(1)

        process(i0 + 1, 1)
        return carry

    lax.fori_loop(0, NBLK // 2, outer, 0)
    drain_out(0)
    drain_out(1)


def kernel(x, token_table, pos_table):
    tT = token_table.T
    xT = jnp.transpose(x, (1, 2, 0))
    packed = _pack_table(tT)
    outT = _gather(xT, packed, pos_table)
    return jnp.transpose(outT, (3, 0, 1, 2))
